# baseline (speedup 1.0000x reference)
import functools

import jax
import jax.numpy as jnp
from jax import lax
from jax.experimental import pallas as pl
from jax.experimental.pallas import tpu as pltpu
from jax.experimental.pallas import tpu_sc as plsc

FACTOR = 128
BATCH = 16384

NC, NS, L = 2, 16, 16
NW = NC * NS
B_PER_W = BATCH // NW
CHUNK = 128
N_CHUNKS = B_PER_W // CHUNK
FC = FACTOR // L

_mesh = plsc.VectorSubcoreMesh(core_axis_name="c", subcore_axis_name="s")


@functools.partial(
    pl.kernel,
    out_type=jax.ShapeDtypeStruct((BATCH,), jnp.float32),
    mesh=_mesh,
    scratch_types=[
        pltpu.VMEM((CHUNK,), jnp.int32),
        pltpu.VMEM((CHUNK,), jnp.int32),
        pltpu.VMEM((CHUNK, FACTOR), jnp.float32),
        pltpu.VMEM((CHUNK, FACTOR), jnp.float32),
        pltpu.VMEM((2 * L,), jnp.float32),
        pltpu.VMEM((B_PER_W + L,), jnp.float32),
        pltpu.VMEM((FACTOR,), jnp.float32),
        pltpu.VMEM((L,), jnp.float32),
        pltpu.SemaphoreType.DMA,
        pltpu.SemaphoreType.DMA,
    ],
)
def _gmf_sc(user_hbm, item_hbm, utab_hbm, itab_hbm, w_hbm, b_hbm, out_hbm,
            idx_u, idx_i, u_rows, i_rows, scr, out_v, w_v, b_v,
            sem_u, sem_i):
    wid = lax.axis_index("s") * NC + lax.axis_index("c")
    base = wid * B_PER_W

    pltpu.sync_copy(w_hbm, w_v)
    pltpu.sync_copy(b_hbm, b_v)
    w_c = [w_v[pl.ds(L * j, L)] for j in range(FC)]
    b_vec = b_v[...]
    scr[pl.ds(L, L)] = jnp.zeros((L,), jnp.float32)

    for c in range(N_CHUNKS):
        cbase = base + c * CHUNK
        pltpu.sync_copy(user_hbm.at[pl.ds(cbase, CHUNK)], idx_u)
        pltpu.sync_copy(item_hbm.at[pl.ds(cbase, CHUNK)], idx_i)
        cp_u = pltpu.async_copy(utab_hbm.at[idx_u], u_rows, sem_u)
        cp_i = pltpu.async_copy(itab_hbm.at[idx_i], i_rows, sem_i)
        cp_u.wait()
        cp_i.wait()

        def body(e, _):
            acc = u_rows[e, pl.ds(0, L)] * i_rows[e, pl.ds(0, L)] * w_c[0]
            for j in range(1, FC):
                acc += u_rows[e, pl.ds(L * j, L)] * i_rows[e, pl.ds(L * j, L)] * w_c[j]
            for d in (8, 4, 2, 1):
                scr[pl.ds(0, L)] = acc
                acc = acc + scr[pl.ds(d, L)]
            out_v[pl.ds(c * CHUNK + e, L)] = acc + b_vec
            return 0

        lax.fori_loop(0, CHUNK, body, 0)

    pltpu.sync_copy(out_v.at[pl.ds(0, B_PER_W)], out_hbm.at[pl.ds(base, B_PER_W)])


def kernel(user, item, user_table, item_table, W, b):
    w_vec = W.reshape(FACTOR)
    b_vec = jnp.broadcast_to(b.reshape(()), (L,))
    return _gmf_sc(user, item, user_table, item_table, w_vec, b_vec)


# speedup vs baseline: 1.3296x; 1.3296x over previous
import functools

import jax
import jax.numpy as jnp
from jax import lax
from jax.experimental import pallas as pl
from jax.experimental.pallas import tpu as pltpu
from jax.experimental.pallas import tpu_sc as plsc

FACTOR = 128
BATCH = 16384

NC, NS, L = 2, 16, 16
NW = NC * NS
B_PER_W = BATCH // NW
CHUNK = 128
N_CHUNKS = B_PER_W // CHUNK
FC = FACTOR // L
BLK = 16
N_BLK = CHUNK // BLK

_mesh = plsc.VectorSubcoreMesh(core_axis_name="c", subcore_axis_name="s")


@functools.partial(
    pl.kernel,
    out_type=jax.ShapeDtypeStruct((BATCH,), jnp.float32),
    mesh=_mesh,
    scratch_types=[
        pltpu.VMEM((2, CHUNK), jnp.int32),
        pltpu.VMEM((2, CHUNK), jnp.int32),
        pltpu.VMEM((2, CHUNK, FACTOR), jnp.float32),
        pltpu.VMEM((2, CHUNK, FACTOR), jnp.float32),
        pltpu.VMEM((BLK * L + L,), jnp.float32),
        pltpu.VMEM((B_PER_W + L,), jnp.float32),
        pltpu.VMEM((FACTOR,), jnp.float32),
        pltpu.VMEM((L,), jnp.float32),
        pltpu.SemaphoreType.DMA,
        pltpu.SemaphoreType.DMA,
        pltpu.SemaphoreType.DMA,
        pltpu.SemaphoreType.DMA,
        pltpu.SemaphoreType.DMA,
        pltpu.SemaphoreType.DMA,
        pltpu.SemaphoreType.DMA,
        pltpu.SemaphoreType.DMA,
    ],
)
def _gmf_sc(user_hbm, item_hbm, utab_hbm, itab_hbm, w_hbm, b_hbm, out_hbm,
            idx_u, idx_i, u_rows, i_rows, acc, out_v, w_v, b_v,
            sxu0, sxu1, sxi0, sxi1, sgu0, sgu1, sgi0, sgi1):
    sxu, sxi = (sxu0, sxu1), (sxi0, sxi1)
    sgu, sgi = (sgu0, sgu1), (sgi0, sgi1)
    wid = lax.axis_index("s") * NC + lax.axis_index("c")
    base = wid * B_PER_W

    pltpu.sync_copy(w_hbm, w_v)
    pltpu.sync_copy(b_hbm, b_v)
    w_c = [w_v[pl.ds(L * j, L)] for j in range(FC)]
    b_vec = b_v[...]

    def issue_idx(c):
        bb = c % 2
        cu = pltpu.async_copy(user_hbm.at[pl.ds(base + c * CHUNK, CHUNK)],
                              idx_u.at[bb], sxu[bb])
        ci = pltpu.async_copy(item_hbm.at[pl.ds(base + c * CHUNK, CHUNK)],
                              idx_i.at[bb], sxi[bb])
        return cu, ci

    def issue_gather(c):
        bb = c % 2
        gu = pltpu.async_copy(utab_hbm.at[idx_u.at[bb]], u_rows.at[bb], sgu[bb])
        gi = pltpu.async_copy(itab_hbm.at[idx_i.at[bb]], i_rows.at[bb], sgi[bb])
        return gu, gi

    idx_cps = {0: issue_idx(0), 1: issue_idx(1)}
    idx_cps[0][0].wait()
    idx_cps[0][1].wait()
    gather_cps = {0: issue_gather(0)}

    for c in range(N_CHUNKS):
        bb = c % 2
        gather_cps[c][0].wait()
        gather_cps[c][1].wait()
        if c + 1 < N_CHUNKS:
            idx_cps[c + 1][0].wait()
            idx_cps[c + 1][1].wait()
            gather_cps[c + 1] = issue_gather(c + 1)
        if c + 2 < N_CHUNKS:
            idx_cps[c + 2] = issue_idx(c + 2)

        ur, ir = u_rows.at[bb], i_rows.at[bb]

        def body(blk, _):
            e0 = blk * BLK
            for m in range(BLK):
                a = ur[e0 + m, pl.ds(0, L)] * ir[e0 + m, pl.ds(0, L)] * w_c[0]
                for j in range(1, FC):
                    a += (ur[e0 + m, pl.ds(L * j, L)]
                          * ir[e0 + m, pl.ds(L * j, L)] * w_c[j])
                acc[pl.ds(m * L, L)] = a
            for d in (8, 4, 2, 1):
                for m in range(BLK):
                    acc[pl.ds(m * L, L)] = (acc[pl.ds(m * L, L)]
                                            + acc[pl.ds(m * L + d, L)])
            for m in range(BLK):
                out_v[pl.ds(c * CHUNK + e0 + m, L)] = acc[pl.ds(m * L, L)] + b_vec
            return 0

        lax.fori_loop(0, N_BLK, body, 0)

    pltpu.sync_copy(out_v.at[pl.ds(0, B_PER_W)], out_hbm.at[pl.ds(base, B_PER_W)])


def kernel(user, item, user_table, item_table, W, b):
    w_vec = W.reshape(FACTOR)
    b_vec = jnp.broadcast_to(b.reshape(()), (L,))
    return _gmf_sc(user, item, user_table, item_table, w_vec, b_vec)
